# initial kernel scaffold (unmeasured)
import jax
import jax.numpy as jnp
from jax import lax
from jax.experimental import pallas as pl
from jax.experimental.pallas import tpu as pltpu

N_DEV = 16
B = 16
H = 16
D = 64
BS = 16
NP = 128
K_TOK = NP * BS
SCALE = D ** -0.5
NEG = -1e30


def kernel(Q, K, V, bt, lens):
    Qs = Q.reshape(B, H, D)
    lens2 = lens.reshape(B, 1)

    def body(q_ref, k_ref, v_ref, bt_ref, lens_ref, out_ref,
             comm_ref, my_ref, send_sems, recv_sems, copy_sem):
        me = lax.axis_index("i")

        bsem = pltpu.get_barrier_semaphore()
        for off in range(1, N_DEV):
            peer = (me + off) % N_DEV
            pl.semaphore_signal(bsem, inc=1, device_id=(peer,),
                                device_id_type=pl.DeviceIdType.MESH)

        q = q_ref[...].astype(jnp.bfloat16)
        k = k_ref[...].astype(jnp.bfloat16).reshape(K_TOK, H, D)
        v = v_ref[...].astype(jnp.bfloat16).reshape(K_TOK, H, D)

        s = lax.dot_general(q, k, (((2,), (2,)), ((1,), (1,))),
                            preferred_element_type=jnp.float32)
        s = s * SCALE

        base = me * NP
        JC = 32
        pid = lax.broadcasted_iota(jnp.int32, (B, JC, K_TOK), 2) // BS + base
        countk = jnp.zeros((B, K_TOK), jnp.float32)
        for j0 in range(0, 128, JC):
            btc = bt_ref[:, j0:j0 + JC]
            jidx = lax.broadcasted_iota(jnp.int32, (B, JC), 1) + j0
            valid = jidx < lens_ref[...]
            hit = (btc[:, :, None] == pid) & valid[:, :, None]
            countk = countk + jnp.sum(jnp.where(hit, 1.0, 0.0), axis=1)

        s = jnp.where((countk > 0)[None], s, NEG)
        m = jnp.max(s, axis=2)
        pw = jnp.exp(s - m[:, :, None]) * countk[None]
        l = jnp.sum(pw, axis=2)
        o = lax.dot_general(pw.astype(jnp.bfloat16), v,
                            (((2,), (0,)), ((0,), (1,))),
                            preferred_element_type=jnp.float32)

        my_ref[0:H, :, :] = o
        my_ref[H, :, 0:B] = m
        my_ref[H, :, B:2 * B] = l

        pl.semaphore_wait(bsem, N_DEV - 1)

        cp = pltpu.make_async_copy(my_ref, comm_ref.at[me], copy_sem)
        cp.start()

        for off in range(1, N_DEV):
            dst = (me + off) % N_DEV
            pltpu.make_async_remote_copy(
                src_ref=my_ref, dst_ref=comm_ref.at[me],
                send_sem=send_sems.at[off], recv_sem=recv_sems.at[me],
                device_id=(dst,), device_id_type=pl.DeviceIdType.MESH,
            ).start()

        cp.wait()
        for off in range(1, N_DEV):
            src = (me + off) % N_DEV
            pltpu.make_async_remote_copy(
                src_ref=my_ref, dst_ref=comm_ref.at[src],
                send_sem=send_sems.at[off], recv_sem=recv_sems.at[src],
                device_id=(src,), device_id_type=pl.DeviceIdType.MESH,
            ).wait_recv()

        c = comm_ref[...]
        o_all = c[:, 0:H]
        m_all = c[:, H, :, 0:B]
        l_all = c[:, H, :, B:2 * B]
        mg = jnp.max(m_all, axis=0)
        scl = jnp.exp(m_all - mg[None])
        den = jnp.sum(scl * l_all, axis=0)
        num = jnp.sum(scl[..., None] * o_all, axis=0)
        res = num / den[..., None]
        out_ref[...] = jnp.transpose(res, (1, 0, 2)).reshape(B, 1, H, D)

        for off in range(1, N_DEV):
            dst = (me + off) % N_DEV
            pltpu.make_async_remote_copy(
                src_ref=my_ref, dst_ref=comm_ref.at[me],
                send_sem=send_sems.at[off], recv_sem=recv_sems.at[me],
                device_id=(dst,), device_id_type=pl.DeviceIdType.MESH,
            ).wait_send()

    return pl.pallas_call(
        body,
        out_shape=jax.ShapeDtypeStruct((B, 1, H, D), jnp.float32),
        in_specs=[pl.BlockSpec(memory_space=pltpu.VMEM)] * 5,
        out_specs=pl.BlockSpec(memory_space=pltpu.VMEM),
        scratch_shapes=[
            pltpu.VMEM((N_DEV, H + 1, 16, 64), jnp.float32),
            pltpu.VMEM((H + 1, 16, 64), jnp.float32),
            pltpu.SemaphoreType.DMA((N_DEV,)),
            pltpu.SemaphoreType.DMA((N_DEV,)),
            pltpu.SemaphoreType.DMA,
        ],
        compiler_params=pltpu.CompilerParams(collective_id=0),
    )(Qs, K, V, bt, lens2)


# baseline (device time: 87408 ns/iter reference)
import jax
import jax.numpy as jnp
from jax import lax
from jax.experimental import pallas as pl
from jax.experimental.pallas import tpu as pltpu

N_DEV = 16
B = 16
H = 16
D = 64
BS = 16
NP = 128
K_TOK = NP * BS
SCALE = D ** -0.5
NEG = -1e30


def kernel(Q, K, V, bt, lens):
    Qs = Q.reshape(B, H, D)
    bt3 = bt.reshape(B, 128, 1)
    lens3 = lens.reshape(B, 1, 1)

    def body(q_ref, k_ref, v_ref, bt_ref, lens_ref, out_ref,
             comm_ref, my_ref, s_buf, ck_buf, send_sems, recv_sems, copy_sem):
        me = lax.axis_index("i")

        bsem = pltpu.get_barrier_semaphore()
        for off in range(1, N_DEV):
            peer = (me + off) % N_DEV
            pl.semaphore_signal(bsem, inc=1, device_id=(peer,),
                                device_id_type=pl.DeviceIdType.MESH)

        base = me * NP
        JC = 32
        pidp = lax.broadcasted_iota(jnp.int32, (B, JC, NP), 2) + base
        jidx = lax.broadcasted_iota(jnp.int32, (B, JC, NP), 1)
        count = jnp.zeros((B, NP), jnp.float32)
        for j0 in range(0, 128, JC):
            btc = bt_ref[:, j0:j0 + JC]
            valid = (jidx + j0) < lens_ref[...]
            hit = (btc == pidp) & valid
            count = count + jnp.sum(jnp.where(hit, 1.0, 0.0), axis=1)

        q = q_ref[...].astype(jnp.bfloat16)
        PC = 32
        C = PC * BS
        pg = lax.broadcasted_iota(jnp.int32, (PC, C), 0)
        tk = lax.broadcasted_iota(jnp.int32, (PC, C), 1) // BS
        expand = jnp.where(pg == tk, 1.0, 0.0).astype(jnp.bfloat16)

        m = jnp.full((H, B), NEG, jnp.float32)
        for p0 in range(0, NP, PC):
            t0 = p0 * BS
            k_c = k_ref[p0:p0 + PC].astype(jnp.bfloat16).reshape(C, H, D)
            s_c = lax.dot_general(q, k_c, (((2,), (2,)), ((1,), (1,))),
                                  preferred_element_type=jnp.float32)
            s_c = s_c * SCALE
            ck_c = lax.dot_general(count[:, p0:p0 + PC].astype(jnp.bfloat16),
                                   expand, (((1,), (0,)), ((), ())),
                                   preferred_element_type=jnp.float32)
            s_c = jnp.where((ck_c > 0)[None], s_c, NEG)
            m = jnp.maximum(m, jnp.max(s_c, axis=2))
            s_buf[:, :, t0:t0 + C] = s_c
            ck_buf[:, t0:t0 + C] = ck_c

        l = jnp.zeros((H, B), jnp.float32)
        o = jnp.zeros((H, B, D), jnp.float32)
        for p0 in range(0, NP, PC):
            t0 = p0 * BS
            v_c = v_ref[p0:p0 + PC].astype(jnp.bfloat16).reshape(C, H, D)
            pw = jnp.exp(s_buf[:, :, t0:t0 + C] - m[:, :, None])
            pw = pw * ck_buf[:, t0:t0 + C][None]
            l = l + jnp.sum(pw, axis=2)
            o = o + lax.dot_general(pw.astype(jnp.bfloat16), v_c,
                                    (((2,), (0,)), ((0,), (1,))),
                                    preferred_element_type=jnp.float32)

        my_ref[0:H, :, :] = o
        my_ref[H, :, 0:B] = m
        my_ref[H, :, B:2 * B] = l

        pl.semaphore_wait(bsem, N_DEV - 1)

        cp = pltpu.make_async_copy(my_ref, comm_ref.at[me], copy_sem)
        cp.start()

        for off in range(1, N_DEV):
            dst = (me + off) % N_DEV
            pltpu.make_async_remote_copy(
                src_ref=my_ref, dst_ref=comm_ref.at[me],
                send_sem=send_sems.at[off], recv_sem=recv_sems.at[me],
                device_id=(dst,), device_id_type=pl.DeviceIdType.MESH,
            ).start()

        cp.wait()
        for off in range(1, N_DEV):
            src = (me + off) % N_DEV
            pltpu.make_async_remote_copy(
                src_ref=my_ref, dst_ref=comm_ref.at[src],
                send_sem=send_sems.at[off], recv_sem=recv_sems.at[src],
                device_id=(src,), device_id_type=pl.DeviceIdType.MESH,
            ).wait_recv()

        c = comm_ref[...]
        o_all = c[:, 0:H]
        m_all = c[:, H, :, 0:B]
        l_all = c[:, H, :, B:2 * B]
        mg = jnp.max(m_all, axis=0)
        scl = jnp.exp(m_all - mg[None])
        den = jnp.sum(scl * l_all, axis=0)
        num = jnp.sum(scl[..., None] * o_all, axis=0)
        res = num / den[..., None]
        out_ref[...] = jnp.transpose(res, (1, 0, 2)).reshape(B, 1, H, D)

        for off in range(1, N_DEV):
            dst = (me + off) % N_DEV
            pltpu.make_async_remote_copy(
                src_ref=my_ref, dst_ref=comm_ref.at[me],
                send_sem=send_sems.at[off], recv_sem=recv_sems.at[me],
                device_id=(dst,), device_id_type=pl.DeviceIdType.MESH,
            ).wait_send()

    return pl.pallas_call(
        body,
        out_shape=jax.ShapeDtypeStruct((B, 1, H, D), jnp.float32),
        in_specs=[pl.BlockSpec(memory_space=pltpu.VMEM)] * 5,
        out_specs=pl.BlockSpec(memory_space=pltpu.VMEM),
        scratch_shapes=[
            pltpu.VMEM((N_DEV, H + 1, 16, 64), jnp.float32),
            pltpu.VMEM((H + 1, 16, 64), jnp.float32),
            pltpu.VMEM((H, B, K_TOK), jnp.float32),
            pltpu.VMEM((B, K_TOK), jnp.float32),
            pltpu.SemaphoreType.DMA((N_DEV,)),
            pltpu.SemaphoreType.DMA((N_DEV,)),
            pltpu.SemaphoreType.DMA,
        ],
        compiler_params=pltpu.CompilerParams(collective_id=0),
    )(Qs, K, V, bt3, lens3)
